# baseline (speedup 1.0000x reference)
import functools

import jax
import jax.numpy as jnp
from jax import lax
from jax.experimental import pallas as pl
from jax.experimental.pallas import tpu as pltpu
from jax.experimental.pallas import tpu_sc as plsc

_RAYS = 4096
_BLK = 2048
_CH = 128


def _shade_body(alb_ref, nrm_ref, ld_ref, lc_ref, w_ref, out_ref):
    f32 = jnp.float32
    ld = ld_ref[...]
    lc = lc_ref[...]
    n = nrm_ref[...]
    a = alb_ref[...]
    w = w_ref[...]
    K = ld.shape[1]
    L = K // 3

    ki = lax.broadcasted_iota(jnp.int32, (3, K), 1)
    ii = lax.broadcasted_iota(jnp.int32, (3, K), 0)
    expand3 = (ki % 3 == ii).astype(f32)
    gk = lax.broadcasted_iota(jnp.int32, (K, L), 0)
    gj = lax.broadcasted_iota(jnp.int32, (K, L), 1)
    group = (gk // 3 == gj).astype(f32)
    uk = lax.broadcasted_iota(jnp.int32, (L, K), 1)
    uj = lax.broadcasted_iota(jnp.int32, (L, K), 0)
    ungroup = (uk // 3 == uj).astype(f32)
    ck = lax.broadcasted_iota(jnp.int32, (K, 3), 0)
    ci = lax.broadcasted_iota(jnp.int32, (K, 3), 1)
    collapse = (ck % 3 == ci).astype(f32)

    n_k = jnp.dot(n, expand3, preferred_element_type=f32)
    dp = jnp.dot(n_k * ld, group, preferred_element_type=f32)
    dp = jnp.clip(dp, 0.0, 1.0)
    cnt = jnp.sum((dp > 0.0).astype(f32), axis=1, keepdims=True)
    cnt = jnp.where(cnt > 0.0, cnt, 1.0)
    dp = dp / cnt
    dp_k = jnp.dot(dp, ungroup, preferred_element_type=f32)
    s3 = jnp.dot(dp_k * lc, collapse, preferred_element_type=f32)
    wrad = a * s3 * w
    out_ref[...] = jnp.concatenate(
        [wrad, w, jnp.zeros_like(ld[:, 0:4])], axis=1)


def _shade(alb, nrm, ld, lc, w):
    S, K = ld.shape
    return pl.pallas_call(
        _shade_body,
        grid=(S // _BLK,),
        in_specs=[
            pl.BlockSpec((_BLK, 3), lambda i: (i, 0)),
            pl.BlockSpec((_BLK, 3), lambda i: (i, 0)),
            pl.BlockSpec((_BLK, K), lambda i: (i, 0)),
            pl.BlockSpec((_BLK, K), lambda i: (i, 0)),
            pl.BlockSpec((_BLK, 1), lambda i: (i, 0)),
        ],
        out_specs=pl.BlockSpec((_BLK, 8), lambda i: (i, 0)),
        out_shape=jax.ShapeDtypeStruct((S, 8), jnp.float32),
    )(alb, nrm, ld, lc, w)


def _segsum(vals, rays, zeros):
    S = vals.shape[0]
    info = plsc.get_sparse_core_info()
    nc, ns = info.num_cores, info.num_subcores
    nw = nc * ns
    per_w = S // nw
    n_ch = per_w // _CH

    vals3 = vals.reshape(nw, per_w, 8)
    rays3 = rays.reshape(nw, n_ch, _CH)
    mesh = plsc.VectorSubcoreMesh(core_axis_name="c", subcore_axis_name="s")

    @functools.partial(
        pl.kernel,
        mesh=mesh,
        compiler_params=pltpu.CompilerParams(use_tc_tiling_on_sc=False),
        out_type=jax.ShapeDtypeStruct((nc, _RAYS, 8), jnp.float32),
        scratch_types=[
            pltpu.VMEM((n_ch, _CH), jnp.int32),
            pltpu.VMEM((per_w, 8), jnp.float32),
            pltpu.VMEM_SHARED((_RAYS, 8), jnp.float32),
            pltpu.SemaphoreType.DMA,
        ],
    )
    def k(vals_hbm, rays_hbm, zeros_hbm, out_hbm, idx_v, vals_v, table_sh, sem):
        c = lax.axis_index("c")
        s = lax.axis_index("s")
        wid = s * nc + c

        @pl.when(s == 0)
        def _zero():
            pltpu.sync_copy(zeros_hbm, table_sh)

        pltpu.sync_copy(rays_hbm.at[wid], idx_v)
        pltpu.sync_copy(vals_hbm.at[wid], vals_v)
        plsc.subcore_barrier()
        for j in range(n_ch):
            pltpu.sync_copy(
                vals_v.at[pl.ds(j * _CH, _CH)],
                table_sh.at[idx_v.at[j]],
                add=True,
            )
        plsc.subcore_barrier()

        @pl.when(s == 0)
        def _flush():
            pltpu.sync_copy(table_sh, out_hbm.at[c])

    return k(vals3, rays3, zeros)


def _finalize_body(pa_ref, pb_ref, out_ref):
    acc = pa_ref[...] + pb_ref[...]
    rgb = acc[:, 0:3]
    wsum = acc[:, 3:4]
    x_safe = jnp.clip(rgb, 1e-4, None)
    srgb = jnp.where(
        rgb <= 0.0031308,
        12.92 * rgb,
        1.055 * jnp.exp(jnp.log(x_safe) * (1.0 / 2.4)) - 0.055,
    )
    out_ref[...] = srgb + (1.0 - wsum)


def _finalize(pa, pb):
    return pl.pallas_call(
        _finalize_body,
        out_shape=jax.ShapeDtypeStruct((_RAYS, 3), jnp.float32),
    )(pa, pb)


def kernel(albedos, normals, light_directions, light_colors, weights,
           ray_indices, num_rays):
    S = light_directions.shape[0]
    K = light_directions.shape[1] * light_directions.shape[2]
    ld = light_directions.reshape(S, K)
    lc = light_colors.reshape(S, K)
    vals = _shade(albedos.reshape(S, 3), normals.reshape(S, 3), ld, lc,
                  weights.reshape(S, 1))
    ridx = (ray_indices.astype(jnp.int32)
            + (jnp.asarray(num_rays, jnp.int32) - _RAYS))
    zeros = jnp.zeros((_RAYS, 8), jnp.float32)
    partials = _segsum(vals, ridx, zeros)
    return _finalize(partials[0], partials[1])


# speedup vs baseline: 3.4255x; 3.4255x over previous
import functools

import jax
import jax.numpy as jnp
from jax import lax
from jax.experimental import pallas as pl
from jax.experimental.pallas import tpu as pltpu
from jax.experimental.pallas import tpu_sc as plsc

_RAYS = 4096
_BLK = 512
_CH = 128


def _shade_body(a_ref, n_ref, ld_ref, lc_ref, w_ref, out_ref):
    f32 = jnp.float32
    ld = ld_ref[...]
    lc = lc_ref[...]
    n = n_ref[...]
    a = a_ref[...]
    w = w_ref[...]

    dp = ld[0] * n[0:1] + ld[1] * n[1:2] + ld[2] * n[2:3]
    dp = jnp.clip(dp, 0.0, 1.0)
    cnt = jnp.sum((dp > 0.0).astype(f32), axis=0, keepdims=True)
    cnt = jnp.where(cnt > 0.0, cnt, 1.0)
    dp = dp / cnt
    rows = []
    for i in range(3):
        si = jnp.sum(dp * lc[i], axis=0, keepdims=True)
        rows.append(a[i:i + 1] * si * w)
    z = jnp.zeros_like(w)
    eight = jnp.concatenate(rows + [w, z, z, z, z], axis=0)
    out_ref[...] = eight.T


def _shade(a_t, n_t, ld_t, lc_t, w_t):
    _, L, S = ld_t.shape
    return pl.pallas_call(
        _shade_body,
        grid=(S // _BLK,),
        in_specs=[
            pl.BlockSpec((3, _BLK), lambda i: (0, i)),
            pl.BlockSpec((3, _BLK), lambda i: (0, i)),
            pl.BlockSpec((3, L, _BLK), lambda i: (0, 0, i)),
            pl.BlockSpec((3, L, _BLK), lambda i: (0, 0, i)),
            pl.BlockSpec((1, _BLK), lambda i: (0, i)),
        ],
        out_specs=pl.BlockSpec((_BLK, 8), lambda i: (i, 0)),
        out_shape=jax.ShapeDtypeStruct((S, 8), jnp.float32),
    )(a_t, n_t, ld_t, lc_t, w_t)


def _segsum(vals, rays, zeros):
    S = vals.shape[0]
    info = plsc.get_sparse_core_info()
    nc, ns = info.num_cores, info.num_subcores
    nw = nc * ns
    per_w = S // nw
    n_ch = per_w // _CH

    vals3 = vals.reshape(nw, per_w, 8)
    rays3 = rays.reshape(nw, n_ch, _CH)
    mesh = plsc.VectorSubcoreMesh(core_axis_name="c", subcore_axis_name="s")

    @functools.partial(
        pl.kernel,
        mesh=mesh,
        compiler_params=pltpu.CompilerParams(use_tc_tiling_on_sc=False),
        out_type=jax.ShapeDtypeStruct((nc, _RAYS, 8), jnp.float32),
        scratch_types=[
            pltpu.VMEM((n_ch, _CH), jnp.int32),
            pltpu.VMEM((per_w, 8), jnp.float32),
            pltpu.VMEM_SHARED((_RAYS, 8), jnp.float32),
            pltpu.SemaphoreType.DMA,
        ],
    )
    def k(vals_hbm, rays_hbm, zeros_hbm, out_hbm, idx_v, vals_v, table_sh, sem):
        c = lax.axis_index("c")
        s = lax.axis_index("s")
        wid = s * nc + c

        @pl.when(s == 0)
        def _zero():
            pltpu.sync_copy(zeros_hbm, table_sh)

        pltpu.sync_copy(rays_hbm.at[wid], idx_v)
        pltpu.sync_copy(vals_hbm.at[wid], vals_v)
        plsc.subcore_barrier()
        for j in range(n_ch):
            pltpu.sync_copy(
                vals_v.at[pl.ds(j * _CH, _CH)],
                table_sh.at[idx_v.at[j]],
                add=True,
            )
        plsc.subcore_barrier()

        @pl.when(s == 0)
        def _flush():
            pltpu.sync_copy(table_sh, out_hbm.at[c])

    return k(vals3, rays3, zeros)


def _finalize_body(pa_ref, pb_ref, out_ref):
    acc = pa_ref[...] + pb_ref[...]
    rgb = acc[:, 0:3]
    wsum = acc[:, 3:4]
    x_safe = jnp.clip(rgb, 1e-4, None)
    srgb = jnp.where(
        rgb <= 0.0031308,
        12.92 * rgb,
        1.055 * jnp.exp(jnp.log(x_safe) * (1.0 / 2.4)) - 0.055,
    )
    out_ref[...] = srgb + (1.0 - wsum)


def _finalize(pa, pb):
    return pl.pallas_call(
        _finalize_body,
        out_shape=jax.ShapeDtypeStruct((_RAYS, 3), jnp.float32),
    )(pa, pb)


def kernel(albedos, normals, light_directions, light_colors, weights,
           ray_indices, num_rays):
    S = light_directions.shape[0]
    ld_t = jnp.transpose(light_directions, (2, 1, 0))
    lc_t = jnp.transpose(light_colors, (2, 1, 0))
    a_t = albedos.T
    n_t = normals.T
    w_t = weights.reshape(S, 1).T
    vals = _shade(a_t, n_t, ld_t, lc_t, w_t)
    ridx = (ray_indices.astype(jnp.int32)
            + (jnp.asarray(num_rays, jnp.int32) - _RAYS))
    zeros = jnp.zeros((_RAYS, 8), jnp.float32)
    partials = _segsum(vals, ridx, zeros)
    return _finalize(partials[0], partials[1])


# speedup vs baseline: 5.3808x; 1.5708x over previous
import functools

import jax
import jax.numpy as jnp
from jax import lax
from jax.experimental import pallas as pl
from jax.experimental.pallas import tpu as pltpu
from jax.experimental.pallas import tpu_sc as plsc

_RAYS = 4096
_BLK = 2048
_CH = 128


def _shade_body(a_ref, n_ref, ld_ref, lc_ref, w_ref, out_ref):
    f32 = jnp.float32
    ld = ld_ref[...]
    lc = lc_ref[...]
    n = n_ref[...]
    a = a_ref[...]
    w = w_ref[...]

    dp = ld[0] * n[0:1] + ld[1] * n[1:2] + ld[2] * n[2:3]
    dp = jnp.clip(dp, 0.0, 1.0)
    cnt = jnp.sum((dp > 0.0).astype(f32), axis=0, keepdims=True)
    cnt = jnp.where(cnt > 0.0, cnt, 1.0)
    dp = dp / cnt
    rows = []
    for i in range(3):
        si = jnp.sum(dp * lc[i], axis=0, keepdims=True)
        rows.append(a[i:i + 1] * si * w)
    z = jnp.zeros_like(w)
    eight = jnp.concatenate(rows + [w, z, z, z, z], axis=0)
    out_ref[...] = eight.T


def _shade(a_t, n_t, ld_t, lc_t, w_t):
    _, L, S = ld_t.shape
    return pl.pallas_call(
        _shade_body,
        grid=(S // _BLK,),
        in_specs=[
            pl.BlockSpec((3, _BLK), lambda i: (0, i)),
            pl.BlockSpec((3, _BLK), lambda i: (0, i)),
            pl.BlockSpec((3, L, _BLK), lambda i: (0, 0, i)),
            pl.BlockSpec((3, L, _BLK), lambda i: (0, 0, i)),
            pl.BlockSpec((1, _BLK), lambda i: (0, i)),
        ],
        out_specs=pl.BlockSpec((_BLK, 8), lambda i: (i, 0)),
        out_shape=jax.ShapeDtypeStruct((S, 8), jnp.float32),
    )(a_t, n_t, ld_t, lc_t, w_t)


def _segsum(vals, rays, zeros):
    S = vals.shape[0]
    info = plsc.get_sparse_core_info()
    nc, ns = info.num_cores, info.num_subcores
    nw = nc * ns
    per_w = S // nw
    n_ch = per_w // _CH

    vals3 = vals.reshape(nw, per_w, 8)
    rays3 = rays.reshape(nw, n_ch, _CH)
    mesh = plsc.VectorSubcoreMesh(core_axis_name="c", subcore_axis_name="s")

    @functools.partial(
        pl.kernel,
        mesh=mesh,
        compiler_params=pltpu.CompilerParams(use_tc_tiling_on_sc=False),
        out_type=jax.ShapeDtypeStruct((nc, _RAYS, 8), jnp.float32),
        scratch_types=[
            pltpu.VMEM((n_ch, _CH), jnp.int32),
            pltpu.VMEM((per_w, 8), jnp.float32),
            pltpu.VMEM_SHARED((_RAYS, 8), jnp.float32),
            pltpu.SemaphoreType.DMA,
        ],
    )
    def k(vals_hbm, rays_hbm, zeros_hbm, out_hbm, idx_v, vals_v, table_sh, sem):
        c = lax.axis_index("c")
        s = lax.axis_index("s")
        wid = s * nc + c

        @pl.when(s == 0)
        def _zero():
            pltpu.sync_copy(zeros_hbm, table_sh)

        pltpu.sync_copy(rays_hbm.at[wid], idx_v)
        pltpu.sync_copy(vals_hbm.at[wid], vals_v)
        plsc.subcore_barrier()
        for j in range(n_ch):
            pltpu.sync_copy(
                vals_v.at[pl.ds(j * _CH, _CH)],
                table_sh.at[idx_v.at[j]],
                add=True,
            )
        plsc.subcore_barrier()

        @pl.when(s == 0)
        def _flush():
            pltpu.sync_copy(table_sh, out_hbm.at[c])

    return k(vals3, rays3, zeros)


def _finalize_body(pa_ref, pb_ref, out_ref):
    acc = pa_ref[...] + pb_ref[...]
    rgb = acc[:, 0:3]
    wsum = acc[:, 3:4]
    x_safe = jnp.clip(rgb, 1e-4, None)
    srgb = jnp.where(
        rgb <= 0.0031308,
        12.92 * rgb,
        1.055 * jnp.exp(jnp.log(x_safe) * (1.0 / 2.4)) - 0.055,
    )
    out_ref[...] = srgb + (1.0 - wsum)


def _finalize(pa, pb):
    return pl.pallas_call(
        _finalize_body,
        out_shape=jax.ShapeDtypeStruct((_RAYS, 3), jnp.float32),
    )(pa, pb)


def kernel(albedos, normals, light_directions, light_colors, weights,
           ray_indices, num_rays):
    S = light_directions.shape[0]
    ld_t = jnp.transpose(light_directions, (2, 1, 0))
    lc_t = jnp.transpose(light_colors, (2, 1, 0))
    a_t = albedos.T
    n_t = normals.T
    w_t = weights.reshape(S, 1).T
    vals = _shade(a_t, n_t, ld_t, lc_t, w_t)
    ridx = (ray_indices.astype(jnp.int32)
            + (jnp.asarray(num_rays, jnp.int32) - _RAYS))
    zeros = jnp.zeros((_RAYS, 8), jnp.float32)
    partials = _segsum(vals, ridx, zeros)
    return _finalize(partials[0], partials[1])


# speedup vs baseline: 5.8393x; 1.0852x over previous
import functools

import jax
import jax.numpy as jnp
from jax import lax
from jax.experimental import pallas as pl
from jax.experimental.pallas import tpu as pltpu
from jax.experimental.pallas import tpu_sc as plsc

_RAYS = 4096
_BLK = 8192
_CH = 128


def _shade_body(a_ref, n_ref, ld_ref, lc_ref, w_ref, out_ref):
    f32 = jnp.float32
    ld = ld_ref[...]
    lc = lc_ref[...]
    n = n_ref[...]
    a = a_ref[...]
    w = w_ref[...]

    dp = ld[0] * n[0:1] + ld[1] * n[1:2] + ld[2] * n[2:3]
    dp = jnp.clip(dp, 0.0, 1.0)
    cnt = jnp.sum((dp > 0.0).astype(f32), axis=0, keepdims=True)
    cnt = jnp.where(cnt > 0.0, cnt, 1.0)
    dp = dp / cnt
    rows = []
    for i in range(3):
        si = jnp.sum(dp * lc[i], axis=0, keepdims=True)
        rows.append(a[i:i + 1] * si * w)
    z = jnp.zeros_like(w)
    eight = jnp.concatenate(rows + [w, z, z, z, z], axis=0)
    out_ref[...] = eight.T


def _shade(a_t, n_t, ld_t, lc_t, w_t):
    _, L, S = ld_t.shape
    return pl.pallas_call(
        _shade_body,
        grid=(S // _BLK,),
        in_specs=[
            pl.BlockSpec((3, _BLK), lambda i: (0, i)),
            pl.BlockSpec((3, _BLK), lambda i: (0, i)),
            pl.BlockSpec((3, L, _BLK), lambda i: (0, 0, i)),
            pl.BlockSpec((3, L, _BLK), lambda i: (0, 0, i)),
            pl.BlockSpec((1, _BLK), lambda i: (0, i)),
        ],
        out_specs=pl.BlockSpec((_BLK, 8), lambda i: (i, 0)),
        out_shape=jax.ShapeDtypeStruct((S, 8), jnp.float32),
    )(a_t, n_t, ld_t, lc_t, w_t)


def _segsum(vals, rays, zeros):
    S = vals.shape[0]
    info = plsc.get_sparse_core_info()
    nc, ns = info.num_cores, info.num_subcores
    nw = nc * ns
    per_w = S // nw
    n_ch = per_w // _CH

    vals3 = vals.reshape(nw, per_w, 8)
    rays3 = rays.reshape(nw, n_ch, _CH)
    mesh = plsc.VectorSubcoreMesh(core_axis_name="c", subcore_axis_name="s")

    @functools.partial(
        pl.kernel,
        mesh=mesh,
        compiler_params=pltpu.CompilerParams(use_tc_tiling_on_sc=False),
        out_type=jax.ShapeDtypeStruct((nc, _RAYS, 8), jnp.float32),
        scratch_types=[
            pltpu.VMEM((n_ch, _CH), jnp.int32),
            pltpu.VMEM((per_w, 8), jnp.float32),
            pltpu.VMEM_SHARED((_RAYS, 8), jnp.float32),
            pltpu.SemaphoreType.DMA,
        ],
    )
    def k(vals_hbm, rays_hbm, zeros_hbm, out_hbm, idx_v, vals_v, table_sh, sem):
        c = lax.axis_index("c")
        s = lax.axis_index("s")
        wid = s * nc + c

        @pl.when(s == 0)
        def _zero():
            pltpu.sync_copy(zeros_hbm, table_sh)

        pltpu.sync_copy(rays_hbm.at[wid], idx_v)
        pltpu.sync_copy(vals_hbm.at[wid], vals_v)
        plsc.subcore_barrier()
        for j in range(n_ch):
            pltpu.sync_copy(
                vals_v.at[pl.ds(j * _CH, _CH)],
                table_sh.at[idx_v.at[j]],
                add=True,
            )
        plsc.subcore_barrier()

        @pl.when(s == 0)
        def _flush():
            pltpu.sync_copy(table_sh, out_hbm.at[c])

    return k(vals3, rays3, zeros)


def _finalize_body(pa_ref, pb_ref, out_ref):
    acc = pa_ref[...] + pb_ref[...]
    rgb = acc[:, 0:3]
    wsum = acc[:, 3:4]
    x_safe = jnp.clip(rgb, 1e-4, None)
    srgb = jnp.where(
        rgb <= 0.0031308,
        12.92 * rgb,
        1.055 * jnp.exp(jnp.log(x_safe) * (1.0 / 2.4)) - 0.055,
    )
    out_ref[...] = srgb + (1.0 - wsum)


def _finalize(pa, pb):
    return pl.pallas_call(
        _finalize_body,
        out_shape=jax.ShapeDtypeStruct((_RAYS, 3), jnp.float32),
    )(pa, pb)


def kernel(albedos, normals, light_directions, light_colors, weights,
           ray_indices, num_rays):
    S = light_directions.shape[0]
    ld_t = jnp.transpose(light_directions, (2, 1, 0))
    lc_t = jnp.transpose(light_colors, (2, 1, 0))
    a_t = albedos.T
    n_t = normals.T
    w_t = weights.reshape(S, 1).T
    vals = _shade(a_t, n_t, ld_t, lc_t, w_t)
    ridx = (ray_indices.astype(jnp.int32)
            + (jnp.asarray(num_rays, jnp.int32) - _RAYS))
    zeros = jnp.zeros((_RAYS, 8), jnp.float32)
    partials = _segsum(vals, ridx, zeros)
    return _finalize(partials[0], partials[1])


# speedup vs baseline: 5.8613x; 1.0038x over previous
import functools

import jax
import jax.numpy as jnp
from jax import lax
from jax.experimental import pallas as pl
from jax.experimental.pallas import tpu as pltpu
from jax.experimental.pallas import tpu_sc as plsc

_RAYS = 4096
_BLK = 8192
_CH = 128


def _shade_body(a_ref, n_ref, ld_ref, lc_ref, w_ref, out_ref):
    f32 = jnp.float32
    ld = ld_ref[...]
    lc = lc_ref[...]
    n = n_ref[...]
    a = a_ref[...]
    w = w_ref[...]

    dp = ld[0] * n[0:1] + ld[1] * n[1:2] + ld[2] * n[2:3]
    dp = jnp.clip(dp, 0.0, 1.0)
    cnt = jnp.sum((dp > 0.0).astype(f32), axis=0, keepdims=True)
    cnt = jnp.where(cnt > 0.0, cnt, 1.0)
    dp = dp * (1.0 / cnt)
    rows = []
    for i in range(3):
        si = jnp.sum(dp * lc[i], axis=0, keepdims=True)
        rows.append(a[i:i + 1] * si * w)
    z = jnp.zeros_like(w)
    eight = jnp.concatenate(rows + [w, z, z, z, z], axis=0)
    out_ref[...] = eight.T


def _shade(a_t, n_t, ld_t, lc_t, w_t):
    _, L, S = ld_t.shape
    return pl.pallas_call(
        _shade_body,
        grid=(S // _BLK,),
        in_specs=[
            pl.BlockSpec((3, _BLK), lambda i: (0, i)),
            pl.BlockSpec((3, _BLK), lambda i: (0, i)),
            pl.BlockSpec((3, L, _BLK), lambda i: (0, 0, i)),
            pl.BlockSpec((3, L, _BLK), lambda i: (0, 0, i)),
            pl.BlockSpec((1, _BLK), lambda i: (0, i)),
        ],
        out_specs=pl.BlockSpec((_BLK, 8), lambda i: (i, 0)),
        out_shape=jax.ShapeDtypeStruct((S, 8), jnp.float32),
    )(a_t, n_t, ld_t, lc_t, w_t)


def _segsum(vals, rays, zeros):
    S = vals.shape[0]
    info = plsc.get_sparse_core_info()
    nc, ns = info.num_cores, info.num_subcores
    nw = nc * ns
    per_w = S // nw
    n_ch = per_w // _CH

    vals3 = vals.reshape(nw, per_w, 8)
    rays3 = rays.reshape(nw, n_ch, _CH)
    mesh = plsc.VectorSubcoreMesh(core_axis_name="c", subcore_axis_name="s")

    @functools.partial(
        pl.kernel,
        mesh=mesh,
        compiler_params=pltpu.CompilerParams(use_tc_tiling_on_sc=False),
        out_type=jax.ShapeDtypeStruct((nc, _RAYS, 8), jnp.float32),
        scratch_types=[
            pltpu.VMEM((n_ch, _CH), jnp.int32),
            pltpu.VMEM((per_w, 8), jnp.float32),
            pltpu.VMEM_SHARED((_RAYS, 8), jnp.float32),
            pltpu.SemaphoreType.DMA,
        ],
    )
    def k(vals_hbm, rays_hbm, zeros_hbm, out_hbm, idx_v, vals_v, table_sh, sem):
        c = lax.axis_index("c")
        s = lax.axis_index("s")
        wid = s * nc + c

        @pl.when(s == 0)
        def _zero():
            pltpu.sync_copy(zeros_hbm, table_sh)

        pltpu.sync_copy(rays_hbm.at[wid], idx_v)
        pltpu.sync_copy(vals_hbm.at[wid], vals_v)
        plsc.subcore_barrier()
        for j in range(n_ch):
            pltpu.sync_copy(
                vals_v.at[pl.ds(j * _CH, _CH)],
                table_sh.at[idx_v.at[j]],
                add=True,
            )
        plsc.subcore_barrier()

        @pl.when(s == 0)
        def _flush():
            pltpu.sync_copy(table_sh, out_hbm.at[c])

    return k(vals3, rays3, zeros)


def _finalize_body(pa_ref, pb_ref, out_ref):
    acc = pa_ref[...] + pb_ref[...]
    rgb = acc[0:3]
    wsum = acc[3:4]
    x_safe = jnp.clip(rgb, 1e-4, None)
    srgb = jnp.where(
        rgb <= 0.0031308,
        12.92 * rgb,
        1.055 * jnp.exp(jnp.log(x_safe) * (1.0 / 2.4)) - 0.055,
    )
    out_ref[...] = srgb + (1.0 - wsum)


def _finalize(pa, pb):
    return pl.pallas_call(
        _finalize_body,
        out_shape=jax.ShapeDtypeStruct((3, _RAYS), jnp.float32),
    )(pa, pb)


def kernel(albedos, normals, light_directions, light_colors, weights,
           ray_indices, num_rays):
    S = light_directions.shape[0]
    ld_t = jnp.transpose(light_directions, (2, 1, 0))
    lc_t = jnp.transpose(light_colors, (2, 1, 0))
    a_t = albedos.T
    n_t = normals.T
    w_t = weights.reshape(S, 1).T
    vals = _shade(a_t, n_t, ld_t, lc_t, w_t)
    ridx = (ray_indices.astype(jnp.int32)
            + (jnp.asarray(num_rays, jnp.int32) - _RAYS))
    zeros = jnp.zeros((_RAYS, 8), jnp.float32)
    partials = _segsum(vals, ridx, zeros)
    return _finalize(partials[0].T, partials[1].T).T
